# baseline (speedup 1.0000x reference)
import functools

import jax
import jax.numpy as jnp
from jax.experimental import pallas as pl


def _add_block_2d(x_ref, pe_ref, o_ref, *, block_r, seq_len):
    i = pl.program_id(0)
    base = (i * block_r) % seq_len
    o_ref[...] = x_ref[...] + pe_ref[pl.ds(base, block_r), :]


@functools.partial(jax.jit, static_argnames=("block_r",))
def _pe_add(x, pe, block_r=1024):
    b, s, d = x.shape
    x2 = x.reshape(b * s, d)
    out = pl.pallas_call(
        functools.partial(_add_block_2d, block_r=block_r, seq_len=s),
        grid=((b * s) // block_r,),
        in_specs=[
            pl.BlockSpec((block_r, d), lambda i: (i, 0)),
            pl.BlockSpec((s, d), lambda i: (0, 0)),
        ],
        out_specs=pl.BlockSpec((block_r, d), lambda i: (i, 0)),
        out_shape=jax.ShapeDtypeStruct((b * s, d), x.dtype),
    )(x2, pe)
    return out.reshape(b, s, d)


def kernel(x, pe):
    return _pe_add(x, pe, block_r=1024)


# speedup vs baseline: 1.0631x; 1.0631x over previous
import functools

import jax
import jax.numpy as jnp
from jax.experimental import pallas as pl


def _add_block_2d(x_ref, pe_ref, o_ref, *, block_r, seq_len):
    i = pl.program_id(0)
    base = (i * block_r) % seq_len
    o_ref[...] = x_ref[...] + pe_ref[pl.ds(base, block_r), :]


@functools.partial(jax.jit, static_argnames=("block_r",))
def _pe_add(x, pe, block_r=1024):
    b, s, d = x.shape
    x2 = x.reshape(b * s, d)
    out = pl.pallas_call(
        functools.partial(_add_block_2d, block_r=block_r, seq_len=s),
        grid=((b * s) // block_r,),
        in_specs=[
            pl.BlockSpec((block_r, d), lambda i: (i, 0)),
            pl.BlockSpec((s, d), lambda i: (0, 0)),
        ],
        out_specs=pl.BlockSpec((block_r, d), lambda i: (i, 0)),
        out_shape=jax.ShapeDtypeStruct((b * s, d), x.dtype),
    )(x2, pe)
    return out.reshape(b, s, d)


def kernel(x, pe):
    return _pe_add(x, pe, block_r=2048)
